# baseline (speedup 1.0000x reference)
import jax
import jax.numpy as jnp
from jax.experimental import pallas as pl
from jax.experimental.pallas import tpu as pltpu

_N_EMB = 32
_ROWS = 3072
_H = 224
_W = 224
_BC = 64
_GRID = (_ROWS // _BC,)


def _body(x_ref, tbl_ref, o_ref):
    sums = jnp.sum(x_ref[...], axis=(1, 2))
    mean = sums[:, None] / float(_H * _W)
    t = jnp.tanh(mean)
    idx = ((t + 1.0) / 2.0 * (_N_EMB - 1)).astype(jnp.int32)
    e = jax.lax.broadcasted_iota(jnp.int32, (1, _N_EMB), 1)
    onehot = (idx == e).astype(jnp.float32)
    o_ref[...] = jnp.sum(onehot * tbl_ref[...], axis=1, keepdims=True)


def kernel(x, beta_table):
    b, c = x.shape[0], x.shape[1]
    x3 = x.reshape(_ROWS, _H, _W)
    tbl = beta_table.reshape(1, _N_EMB)
    out = pl.pallas_call(
        _body,
        grid=_GRID,
        in_specs=[
            pl.BlockSpec((_BC, _H, _W), lambda i: (i, 0, 0)),
            pl.BlockSpec((1, _N_EMB), lambda i: (0, 0)),
        ],
        out_specs=pl.BlockSpec((_BC, 1), lambda i: (i, 0)),
        out_shape=jax.ShapeDtypeStruct((_ROWS, 1), jnp.float32),
        compiler_params=pltpu.CompilerParams(
            dimension_semantics=("parallel",),
        ),
    )(x3, tbl)
    return out.reshape(b, c, 1, 1)


# speedup vs baseline: 1.0001x; 1.0001x over previous
import jax
import jax.numpy as jnp
from jax.experimental import pallas as pl
from jax.experimental.pallas import tpu as pltpu

_N_EMB = 32
_ROWS = 3072
_H = 224
_W = 224
_NS = 4
_BC = 32
_SEG = _ROWS // _NS
_GRID = (_SEG // _BC,)


def _body(*refs):
    x_refs = refs[:_NS]
    tbl_ref = refs[_NS]
    o_ref = refs[_NS + 1]
    e = jax.lax.broadcasted_iota(jnp.int32, (1, _N_EMB), 1)
    for k in range(_NS):
        sums = jnp.sum(x_refs[k][...], axis=(1, 2))
        mean = sums[:, None] / float(_H * _W)
        t = jnp.tanh(mean)
        idx = ((t + 1.0) / 2.0 * (_N_EMB - 1)).astype(jnp.int32)
        onehot = (idx == e).astype(jnp.float32)
        o_ref[k, :, :] = jnp.sum(onehot * tbl_ref[...], axis=1, keepdims=True)


def _x_spec(k):
    nblk = _SEG // _BC
    return pl.BlockSpec((_BC, _H, _W), lambda i, k=k, n=nblk: (k * n + i, 0, 0))


def kernel(x, beta_table):
    b, c = x.shape[0], x.shape[1]
    x3 = x.reshape(_ROWS, _H, _W)
    tbl = beta_table.reshape(1, _N_EMB)
    out = pl.pallas_call(
        _body,
        grid=_GRID,
        in_specs=[_x_spec(k) for k in range(_NS)]
        + [pl.BlockSpec((1, _N_EMB), lambda i: (0, 0))],
        out_specs=pl.BlockSpec((_NS, _BC, 1), lambda i: (0, i, 0)),
        out_shape=jax.ShapeDtypeStruct((_NS, _SEG, 1), jnp.float32),
        compiler_params=pltpu.CompilerParams(
            dimension_semantics=("parallel",),
        ),
    )(*([x3] * _NS), tbl)
    return out.reshape(b, c, 1, 1)


# speedup vs baseline: 1.0002x; 1.0001x over previous
import jax
import jax.numpy as jnp
from jax.experimental import pallas as pl
from jax.experimental.pallas import tpu as pltpu

_N_EMB = 32
_ROWS = 3072
_H = 224
_W = 224
_CK = 16
_NBUF = 4
_PER_STEP = 8
_BC = _CK * _PER_STEP
_NSTEP = _ROWS // _BC
_NCHUNK = _ROWS // _CK


def _body(x_hbm, tbl_ref, o_ref, ring, sems):
    i = pl.program_id(0)
    e = jax.lax.broadcasted_iota(jnp.int32, (1, _N_EMB), 1)

    def issue(gid, slot):
        pltpu.make_async_copy(
            x_hbm.at[pl.ds(gid * _CK, _CK)], ring.at[slot], sems.at[slot]
        ).start()

    @pl.when(i == 0)
    def _prime():
        for c in range(_NBUF):
            issue(jnp.int32(c), c)

    for c in range(_PER_STEP):
        slot = c % _NBUF
        gid = i * _PER_STEP + c
        pltpu.make_async_copy(
            x_hbm.at[pl.ds(gid * _CK, _CK)], ring.at[slot], sems.at[slot]
        ).wait()
        sums = jnp.sum(ring[slot], axis=(1, 2))
        mean = sums[:, None] / float(_H * _W)
        t = jnp.tanh(mean)
        idx = ((t + 1.0) / 2.0 * (_N_EMB - 1)).astype(jnp.int32)
        onehot = (idx == e).astype(jnp.float32)
        o_ref[c * _CK:(c + 1) * _CK, :] = jnp.sum(
            onehot * tbl_ref[...], axis=1, keepdims=True
        )
        if c < _PER_STEP - _NBUF:
            issue(gid + _NBUF, slot)
        else:
            @pl.when(i < _NSTEP - 1)
            def _refill():
                issue(gid + _NBUF, slot)


def kernel(x, beta_table):
    b, c = x.shape[0], x.shape[1]
    x3 = x.reshape(_ROWS, _H, _W)
    tbl = beta_table.reshape(1, _N_EMB)
    out = pl.pallas_call(
        _body,
        grid=(_NSTEP,),
        in_specs=[
            pl.BlockSpec(memory_space=pltpu.MemorySpace.HBM),
            pl.BlockSpec((1, _N_EMB), lambda i: (0, 0)),
        ],
        out_specs=pl.BlockSpec((_BC, 1), lambda i: (i, 0)),
        out_shape=jax.ShapeDtypeStruct((_ROWS, 1), jnp.float32),
        scratch_shapes=[
            pltpu.VMEM((_NBUF, _CK, _H, _W), jnp.float32),
            pltpu.SemaphoreType.DMA((_NBUF,)),
        ],
        compiler_params=pltpu.CompilerParams(
            dimension_semantics=("arbitrary",),
        ),
    )(x3, tbl)
    return out.reshape(b, c, 1, 1)
